# baseline (speedup 1.0000x reference)
import jax
import jax.numpy as jnp
from jax.experimental import pallas as pl
from jax.experimental.pallas import tpu as pltpu

_SCORE_THRESH = 0.05
_NMS_THRESH = 0.5
_MAX_DET = 100
_N = 20000
_R = 160
_C = 128
_PAD = _R * _C
_NBLK = _R // 8
_NEG = -jnp.inf


def _nms_kernel(x1_ref, y1_ref, x2_ref, y2_ref, s_ref, out_ref, work_ref, area_ref):
    area_ref[...] = (x2_ref[...] - x1_ref[...]) * (y2_ref[...] - y1_ref[...])
    s = s_ref[...]
    work_ref[...] = jnp.where(s > _SCORE_THRESH, s, _NEG)

    lane = jax.lax.broadcasted_iota(jnp.int32, (1, _C), 1)
    row8 = jax.lax.broadcasted_iota(jnp.int32, (8, _C), 0)

    def step(i, carry):
        bval, bx1, by1, bx2, by2 = carry
        valid = bval != _NEG
        barea = (bx2 - bx1) * (by2 - by1)
        acc_v = jnp.full((8, _C), _NEG, dtype=jnp.float32)
        acc_r = jnp.zeros((8, _C), dtype=jnp.int32)
        acc_1 = jnp.zeros((8, _C), dtype=jnp.float32)
        acc_2 = acc_1
        acc_3 = acc_1
        acc_4 = acc_1
        for v in range(_NBLK):
            sl = pl.ds(v * 8, 8)
            xv1 = x1_ref[sl, :]
            yv1 = y1_ref[sl, :]
            xv2 = x2_ref[sl, :]
            yv2 = y2_ref[sl, :]
            wv = work_ref[sl, :]
            xx1 = jnp.maximum(xv1, bx1)
            yy1 = jnp.maximum(yv1, by1)
            xx2 = jnp.minimum(xv2, bx2)
            yy2 = jnp.minimum(yv2, by2)
            inter = jnp.maximum(xx2 - xx1, 0.0) * jnp.maximum(yy2 - yy1, 0.0)
            iou = inter / (area_ref[sl, :] + barea - inter + 1e-9)
            nw = jnp.where((iou > _NMS_THRESH) & valid, _NEG, wv)
            work_ref[sl, :] = nw
            gt = nw > acc_v
            acc_r = jnp.where(gt, row8 + v * 8, acc_r)
            acc_1 = jnp.where(gt, xv1, acc_1)
            acc_2 = jnp.where(gt, yv1, acc_2)
            acc_3 = jnp.where(gt, xv2, acc_3)
            acc_4 = jnp.where(gt, yv2, acc_4)
            acc_v = jnp.where(gt, nw, acc_v)
        for k in (4, 2, 1):
            rv = pltpu.roll(acc_v, k, 0)
            rr = pltpu.roll(acc_r, k, 0)
            b = (rv > acc_v) | ((rv == acc_v) & (rr < acc_r))
            acc_v = jnp.where(b, rv, acc_v)
            acc_r = jnp.where(b, rr, acc_r)
            acc_1 = jnp.where(b, pltpu.roll(acc_1, k, 0), acc_1)
            acc_2 = jnp.where(b, pltpu.roll(acc_2, k, 0), acc_2)
            acc_3 = jnp.where(b, pltpu.roll(acc_3, k, 0), acc_3)
            acc_4 = jnp.where(b, pltpu.roll(acc_4, k, 0), acc_4)
        colv = acc_v[0:1, :]
        colr = acc_r[0:1, :]
        c1 = acc_1[0:1, :]
        c2 = acc_2[0:1, :]
        c3 = acc_3[0:1, :]
        c4 = acc_4[0:1, :]
        m1 = jnp.max(colv, axis=1, keepdims=True)
        sel = colv == m1
        flatf = (colr * _C + lane).astype(jnp.float32)
        pk = jnp.concatenate(
            [
                jnp.where(sel, -flatf, _NEG),
                jnp.where(sel, c1, _NEG),
                jnp.where(sel, c2, _NEG),
                jnp.where(sel, c3, _NEG),
                jnp.where(sel, c4, _NEG),
                jnp.where(sel, flatf, _NEG),
                jnp.where(sel, flatf, _NEG),
                jnp.where(sel, flatf, _NEG),
            ],
            axis=0,
        )
        r8 = jnp.max(pk, axis=1, keepdims=True)
        negf = r8[0:1, :]
        fx1 = r8[1:2, :]
        fy1 = r8[2:3, :]
        fx2 = r8[3:4, :]
        fy2 = r8[4:5, :]
        posf = r8[5:6, :]
        tie_s = (posf + negf)[0, 0] != 0.0

        def fix():
            um = flatf == -negf
            pk2 = jnp.concatenate(
                [
                    jnp.where(um, c1, _NEG),
                    jnp.where(um, c2, _NEG),
                    jnp.where(um, c3, _NEG),
                    jnp.where(um, c4, _NEG),
                ]
                + [jnp.where(um, c4, _NEG)] * 4,
                axis=0,
            )
            q8 = jnp.max(pk2, axis=1, keepdims=True)
            return q8[0:1, :], q8[1:2, :], q8[2:3, :], q8[3:4, :]

        nx1, ny1, nx2, ny2 = jax.lax.cond(
            tie_s, fix, lambda: (fx1, fy1, fx2, fy2)
        )
        nvalid = m1 != _NEG
        rowv = (
            jnp.where(lane == 0, nx1, 0.0)
            + jnp.where(lane == 1, ny1, 0.0)
            + jnp.where(lane == 2, nx2, 0.0)
            + jnp.where(lane == 3, ny2, 0.0)
            + jnp.where(lane == 4, m1, 0.0)
        )
        out_ref[pl.ds(i, 1), :] = jnp.where(nvalid, rowv, 0.0)
        return (m1, nx1, ny1, nx2, ny2)

    z = jnp.zeros((1, 1), dtype=jnp.float32)
    init = (jnp.full((1, 1), _NEG, dtype=jnp.float32), z, z, z, z)
    jax.lax.fori_loop(0, _MAX_DET, step, init)


def kernel(boxes, scores):
    pad_boxes = jnp.zeros((_PAD - _N, 4), dtype=boxes.dtype)
    b = jnp.concatenate([boxes, pad_boxes], axis=0)
    s = jnp.concatenate(
        [scores, jnp.full((_PAD - _N,), -1.0, dtype=scores.dtype)], axis=0
    ).reshape(_R, _C)
    x1 = b[:, 0].reshape(_R, _C)
    y1 = b[:, 1].reshape(_R, _C)
    x2 = b[:, 2].reshape(_R, _C)
    y2 = b[:, 3].reshape(_R, _C)
    out = pl.pallas_call(
        _nms_kernel,
        out_shape=jax.ShapeDtypeStruct((_MAX_DET, _C), jnp.float32),
        scratch_shapes=[
            pltpu.VMEM((_R, _C), jnp.float32),
            pltpu.VMEM((_R, _C), jnp.float32),
        ],
    )(x1, y1, x2, y2, s)
    return out[:, :5]


# speedup vs baseline: 1.3996x; 1.3996x over previous
import jax
import jax.numpy as jnp
from jax.experimental import pallas as pl
from jax.experimental.pallas import tpu as pltpu

_SCORE_THRESH = 0.05
_NMS_THRESH = 0.5
_MAX_DET = 100
_N = 20000
_R = 160
_C = 128
_PAD = _R * _C
_NBLK = _R // 8
_NEG = -jnp.inf


def _nms_kernel(x1_ref, y1_ref, x2_ref, y2_ref, s_ref, out_ref, work_ref, area_ref):
    area_ref[...] = (x2_ref[...] - x1_ref[...]) * (y2_ref[...] - y1_ref[...])
    s = s_ref[...]
    work_ref[...] = jnp.where(s > _SCORE_THRESH, s, _NEG)

    lane = jax.lax.broadcasted_iota(jnp.int32, (1, _C), 1)
    row8 = jax.lax.broadcasted_iota(jnp.int32, (8, _C), 0)
    lanef = lane.astype(jnp.float32)

    def iou_of(ax1, ay1, ax2, ay2, bx1, by1, bx2, by2):
        xx1 = jnp.maximum(ax1, bx1)
        yy1 = jnp.maximum(ay1, by1)
        xx2 = jnp.minimum(ax2, bx2)
        yy2 = jnp.minimum(ay2, by2)
        inter = jnp.maximum(xx2 - xx1, 0.0) * jnp.maximum(yy2 - yy1, 0.0)
        aarea = (ax2 - ax1) * (ay2 - ay1)
        barea = (bx2 - bx1) * (by2 - by1)
        return inter / (aarea + barea - inter + 1e-9)

    def round_body(carry):
        (n_out, b1v, p11, p12, p13, p14, b2v, p21, p22, p23, p24) = carry
        valid1 = b1v != _NEG
        valid2 = b2v != _NEG
        barea1 = (p13 - p11) * (p14 - p12)
        barea2 = (p23 - p21) * (p24 - p22)
        acc_v = jnp.full((8, _C), _NEG, dtype=jnp.float32)
        acc_r = jnp.zeros((8, _C), dtype=jnp.int32)
        acc_v2 = acc_v
        acc_r2 = acc_r
        acc_1 = jnp.zeros((8, _C), dtype=jnp.float32)
        acc_2 = acc_1
        acc_3 = acc_1
        acc_4 = acc_1
        for v in range(_NBLK):
            sl = pl.ds(v * 8, 8)
            xv1 = x1_ref[sl, :]
            yv1 = y1_ref[sl, :]
            xv2 = x2_ref[sl, :]
            yv2 = y2_ref[sl, :]
            wv = work_ref[sl, :]
            av = area_ref[sl, :]
            xxa = jnp.maximum(xv1, p11)
            yya = jnp.maximum(yv1, p12)
            xxb = jnp.minimum(xv2, p13)
            yyb = jnp.minimum(yv2, p14)
            ia = jnp.maximum(xxb - xxa, 0.0) * jnp.maximum(yyb - yya, 0.0)
            iou1 = ia / (av + barea1 - ia + 1e-9)
            sup = (iou1 > _NMS_THRESH) & valid1
            xxa = jnp.maximum(xv1, p21)
            yya = jnp.maximum(yv1, p22)
            xxb = jnp.minimum(xv2, p23)
            yyb = jnp.minimum(yv2, p24)
            ib = jnp.maximum(xxb - xxa, 0.0) * jnp.maximum(yyb - yya, 0.0)
            iou2 = ib / (av + barea2 - ib + 1e-9)
            sup = sup | ((iou2 > _NMS_THRESH) & valid2)
            nw = jnp.where(sup, _NEG, wv)
            work_ref[sl, :] = nw
            gt1 = nw > acc_v
            gt2 = (~gt1) & (nw > acc_v2)
            acc_v2 = jnp.where(gt1, acc_v, jnp.where(gt2, nw, acc_v2))
            acc_r2 = jnp.where(gt1, acc_r, jnp.where(gt2, row8 + v * 8, acc_r2))
            acc_r = jnp.where(gt1, row8 + v * 8, acc_r)
            acc_1 = jnp.where(gt1, xv1, acc_1)
            acc_2 = jnp.where(gt1, yv1, acc_2)
            acc_3 = jnp.where(gt1, xv2, acc_3)
            acc_4 = jnp.where(gt1, yv2, acc_4)
            acc_v = jnp.where(gt1, nw, acc_v)
        for k in (4, 2, 1):
            bv = pltpu.roll(acc_v, k, 0)
            br = pltpu.roll(acc_r, k, 0)
            bv2 = pltpu.roll(acc_v2, k, 0)
            br2 = pltpu.roll(acc_r2, k, 0)
            bfirst = (bv > acc_v) | ((bv == acc_v) & (br < acc_r))
            tx = (bv2 > acc_v) | ((bv2 == acc_v) & (br2 < acc_r))
            vx = jnp.where(tx, bv2, acc_v)
            rx = jnp.where(tx, br2, acc_r)
            ty = (bv > acc_v2) | ((bv == acc_v2) & (br < acc_r2))
            vy = jnp.where(ty, bv, acc_v2)
            ry = jnp.where(ty, br, acc_r2)
            acc_v2 = jnp.where(bfirst, vx, vy)
            acc_r2 = jnp.where(bfirst, rx, ry)
            acc_v = jnp.where(bfirst, bv, acc_v)
            acc_r = jnp.where(bfirst, br, acc_r)
            acc_1 = jnp.where(bfirst, pltpu.roll(acc_1, k, 0), acc_1)
            acc_2 = jnp.where(bfirst, pltpu.roll(acc_2, k, 0), acc_2)
            acc_3 = jnp.where(bfirst, pltpu.roll(acc_3, k, 0), acc_3)
            acc_4 = jnp.where(bfirst, pltpu.roll(acc_4, k, 0), acc_4)
        colv = acc_v[0:1, :]
        colv2 = acc_v2[0:1, :]
        c1 = acc_1[0:1, :]
        c2 = acc_2[0:1, :]
        c3 = acc_3[0:1, :]
        c4 = acc_4[0:1, :]
        flat1 = (acc_r[0:1, :] * _C + lane).astype(jnp.float32)
        flat2 = (acc_r2[0:1, :] * _C + lane).astype(jnp.float32)
        m1 = jnp.max(colv, axis=1, keepdims=True)
        sel = colv == m1
        pk = jnp.concatenate(
            [
                jnp.where(sel, -flat1, _NEG),
                jnp.where(sel, c1, _NEG),
                jnp.where(sel, c2, _NEG),
                jnp.where(sel, c3, _NEG),
                jnp.where(sel, c4, _NEG),
                jnp.where(sel, flat1, _NEG),
                jnp.where(sel, flat1, _NEG),
                jnp.where(sel, flat1, _NEG),
            ],
            axis=0,
        )
        r8 = jnp.max(pk, axis=1, keepdims=True)
        negf = r8[0:1, :]
        fx1 = r8[1:2, :]
        fy1 = r8[2:3, :]
        fx2 = r8[3:4, :]
        fy2 = r8[4:5, :]
        posf = r8[5:6, :]
        mixv = jnp.where(sel, colv2, colv)
        m2 = jnp.max(mixv, axis=1, keepdims=True)
        sel2 = mixv == m2
        mixf = jnp.where(sel, flat2, flat1)
        pk2 = jnp.concatenate(
            [
                jnp.where(sel2, -mixf, _NEG),
                jnp.where(sel2 & (~sel), c1, _NEG),
                jnp.where(sel2 & (~sel), c2, _NEG),
                jnp.where(sel2 & (~sel), c3, _NEG),
                jnp.where(sel2 & (~sel), c4, _NEG),
                jnp.where(sel2, mixf, _NEG),
                jnp.where(sel2, mixf, _NEG),
                jnp.where(sel2, mixf, _NEG),
            ],
            axis=0,
        )
        q8 = jnp.max(pk2, axis=1, keepdims=True)
        negf2 = q8[0:1, :]
        sx1 = q8[1:2, :]
        sy1 = q8[2:3, :]
        sx2 = q8[3:4, :]
        sy2 = q8[4:5, :]
        posf2 = q8[5:6, :]
        tie1v = (posf + negf) != 0.0
        tie2v = (posf2 + negf2) != 0.0
        wlane = -negf - jnp.floor(-negf / _C) * _C
        slane_ = -negf2 - jnp.floor(-negf2 / _C) * _C
        samelane = wlane == slane_
        iou12 = iou_of(sx1, sy1, sx2, sy2, fx1, fy1, fx2, fy2)
        sup2 = iou12 > _NMS_THRESH
        bad2v = tie1v | tie2v | samelane | (m2 == _NEG) | sup2
        code = jnp.where(tie1v, 1.0, 0.0) + jnp.where(bad2v, 2.0, 0.0)
        code_s = code[0, 0]
        tie1_s = (code_s == 1.0) | (code_s == 3.0)
        accept2_s = (code_s < 2.0) & (n_out < _MAX_DET - 1)

        def fixco():
            um = flat1 == -negf
            fpk = jnp.concatenate(
                [
                    jnp.where(um, c1, _NEG),
                    jnp.where(um, c2, _NEG),
                    jnp.where(um, c3, _NEG),
                    jnp.where(um, c4, _NEG),
                ]
                + [jnp.where(um, c4, _NEG)] * 4,
                axis=0,
            )
            f8 = jnp.max(fpk, axis=1, keepdims=True)
            return f8[0:1, :], f8[1:2, :], f8[2:3, :], f8[3:4, :]

        wx1, wy1, wx2, wy2 = jax.lax.cond(
            tie1_s, fixco, lambda: (fx1, fy1, fx2, fy2)
        )
        valid_n = m1 != _NEG
        rowv1 = (
            jnp.where(lane == 0, wx1, 0.0)
            + jnp.where(lane == 1, wy1, 0.0)
            + jnp.where(lane == 2, wx2, 0.0)
            + jnp.where(lane == 3, wy2, 0.0)
            + jnp.where(lane == 4, m1, 0.0)
        )
        out_ref[pl.ds(n_out, 1), :] = jnp.where(valid_n, rowv1, 0.0)

        @pl.when(accept2_s)
        def _():
            rowv2 = (
                jnp.where(lane == 0, sx1, 0.0)
                + jnp.where(lane == 1, sy1, 0.0)
                + jnp.where(lane == 2, sx2, 0.0)
                + jnp.where(lane == 3, sy2, 0.0)
                + jnp.where(lane == 4, m2, 0.0)
            )
            out_ref[pl.ds(n_out + 1, 1), :] = rowv2

        n_next = n_out + jnp.where(accept2_s, 2, 1).astype(jnp.int32)
        nb2v = jnp.where(accept2_s, m2, _NEG)
        return (n_next, m1, wx1, wy1, wx2, wy2, nb2v, sx1, sy1, sx2, sy2)

    z = jnp.zeros((1, 1), dtype=jnp.float32)
    ninf = jnp.full((1, 1), _NEG, dtype=jnp.float32)
    jax.lax.while_loop(
        lambda c: c[0] < _MAX_DET,
        round_body,
        (jnp.int32(0), ninf, z, z, z, z, ninf, z, z, z, z),
    )


def kernel(boxes, scores):
    pad_boxes = jnp.zeros((_PAD - _N, 4), dtype=boxes.dtype)
    b = jnp.concatenate([boxes, pad_boxes], axis=0)
    s = jnp.concatenate(
        [scores, jnp.full((_PAD - _N,), -1.0, dtype=scores.dtype)], axis=0
    ).reshape(_R, _C)
    x1 = b[:, 0].reshape(_R, _C)
    y1 = b[:, 1].reshape(_R, _C)
    x2 = b[:, 2].reshape(_R, _C)
    y2 = b[:, 3].reshape(_R, _C)
    out = pl.pallas_call(
        _nms_kernel,
        out_shape=jax.ShapeDtypeStruct((_MAX_DET, _C), jnp.float32),
        scratch_shapes=[
            pltpu.VMEM((_R, _C), jnp.float32),
            pltpu.VMEM((_R, _C), jnp.float32),
        ],
    )(x1, y1, x2, y2, s)
    return out[:, :5]


# speedup vs baseline: 1.5164x; 1.0834x over previous
import jax
import jax.numpy as jnp
from jax.experimental import pallas as pl
from jax.experimental.pallas import tpu as pltpu

_SCORE_THRESH = 0.05
_NMS_THRESH = 0.5
_MAX_DET = 100
_N = 20000
_R = 160
_C = 128
_PAD = _R * _C
_NBLK = _R // 8
_NEG = -jnp.inf


def _nms_kernel(x1_ref, y1_ref, x2_ref, y2_ref, s_ref, out_ref, work_ref, area_ref):
    area_ref[...] = (x2_ref[...] - x1_ref[...]) * (y2_ref[...] - y1_ref[...])
    s = s_ref[...]
    work_ref[...] = jnp.where(s > _SCORE_THRESH, s, _NEG)

    lane = jax.lax.broadcasted_iota(jnp.int32, (1, _C), 1)
    row8 = jax.lax.broadcasted_iota(jnp.int32, (8, _C), 0)
    lanef = lane.astype(jnp.float32)

    def iou_of(ax1, ay1, ax2, ay2, bx1, by1, bx2, by2):
        xx1 = jnp.maximum(ax1, bx1)
        yy1 = jnp.maximum(ay1, by1)
        xx2 = jnp.minimum(ax2, bx2)
        yy2 = jnp.minimum(ay2, by2)
        inter = jnp.maximum(xx2 - xx1, 0.0) * jnp.maximum(yy2 - yy1, 0.0)
        aarea = (ax2 - ax1) * (ay2 - ay1)
        barea = (bx2 - bx1) * (by2 - by1)
        return inter / (aarea + barea - inter + 1e-9)

    def round_body(carry):
        (n_out, b1v, p11, p12, p13, p14, b2v, p21, p22, p23, p24,
         b3v, p31, p32, p33, p34) = carry
        valid1 = b1v != _NEG
        valid2 = b2v != _NEG
        valid3 = b3v != _NEG
        barea1 = (p13 - p11) * (p14 - p12)
        barea2 = (p23 - p21) * (p24 - p22)
        barea3 = (p33 - p31) * (p34 - p32)
        acc_v = jnp.full((8, _C), _NEG, dtype=jnp.float32)
        acc_r = jnp.zeros((8, _C), dtype=jnp.int32)
        acc_v2 = acc_v
        acc_r2 = acc_r
        acc_1 = jnp.zeros((8, _C), dtype=jnp.float32)
        acc_2 = acc_1
        acc_3 = acc_1
        acc_4 = acc_1
        for v in range(_NBLK):
            sl = pl.ds(v * 8, 8)
            xv1 = x1_ref[sl, :]
            yv1 = y1_ref[sl, :]
            xv2 = x2_ref[sl, :]
            yv2 = y2_ref[sl, :]
            wv = work_ref[sl, :]
            av = area_ref[sl, :]
            xxa = jnp.maximum(xv1, p11)
            yya = jnp.maximum(yv1, p12)
            xxb = jnp.minimum(xv2, p13)
            yyb = jnp.minimum(yv2, p14)
            ia = jnp.maximum(xxb - xxa, 0.0) * jnp.maximum(yyb - yya, 0.0)
            iou1 = ia / (av + barea1 - ia + 1e-9)
            sup = (iou1 > _NMS_THRESH) & valid1
            xxa = jnp.maximum(xv1, p21)
            yya = jnp.maximum(yv1, p22)
            xxb = jnp.minimum(xv2, p23)
            yyb = jnp.minimum(yv2, p24)
            ib = jnp.maximum(xxb - xxa, 0.0) * jnp.maximum(yyb - yya, 0.0)
            iou2 = ib / (av + barea2 - ib + 1e-9)
            sup = sup | ((iou2 > _NMS_THRESH) & valid2)
            xxa = jnp.maximum(xv1, p31)
            yya = jnp.maximum(yv1, p32)
            xxb = jnp.minimum(xv2, p33)
            yyb = jnp.minimum(yv2, p34)
            ic = jnp.maximum(xxb - xxa, 0.0) * jnp.maximum(yyb - yya, 0.0)
            iou3 = ic / (av + barea3 - ic + 1e-9)
            sup = sup | ((iou3 > _NMS_THRESH) & valid3)
            nw = jnp.where(sup, _NEG, wv)
            work_ref[sl, :] = nw
            gt1 = nw > acc_v
            gt2 = (~gt1) & (nw > acc_v2)
            acc_v2 = jnp.where(gt1, acc_v, jnp.where(gt2, nw, acc_v2))
            acc_r2 = jnp.where(gt1, acc_r, jnp.where(gt2, row8 + v * 8, acc_r2))
            acc_r = jnp.where(gt1, row8 + v * 8, acc_r)
            acc_1 = jnp.where(gt1, xv1, acc_1)
            acc_2 = jnp.where(gt1, yv1, acc_2)
            acc_3 = jnp.where(gt1, xv2, acc_3)
            acc_4 = jnp.where(gt1, yv2, acc_4)
            acc_v = jnp.where(gt1, nw, acc_v)
        for k in (4, 2, 1):
            bv = pltpu.roll(acc_v, k, 0)
            br = pltpu.roll(acc_r, k, 0)
            bv2 = pltpu.roll(acc_v2, k, 0)
            br2 = pltpu.roll(acc_r2, k, 0)
            bfirst = (bv > acc_v) | ((bv == acc_v) & (br < acc_r))
            tx = (bv2 > acc_v) | ((bv2 == acc_v) & (br2 < acc_r))
            vx = jnp.where(tx, bv2, acc_v)
            rx = jnp.where(tx, br2, acc_r)
            ty = (bv > acc_v2) | ((bv == acc_v2) & (br < acc_r2))
            vy = jnp.where(ty, bv, acc_v2)
            ry = jnp.where(ty, br, acc_r2)
            acc_v2 = jnp.where(bfirst, vx, vy)
            acc_r2 = jnp.where(bfirst, rx, ry)
            acc_v = jnp.where(bfirst, bv, acc_v)
            acc_r = jnp.where(bfirst, br, acc_r)
            acc_1 = jnp.where(bfirst, pltpu.roll(acc_1, k, 0), acc_1)
            acc_2 = jnp.where(bfirst, pltpu.roll(acc_2, k, 0), acc_2)
            acc_3 = jnp.where(bfirst, pltpu.roll(acc_3, k, 0), acc_3)
            acc_4 = jnp.where(bfirst, pltpu.roll(acc_4, k, 0), acc_4)
        colv = acc_v[0:1, :]
        colv2 = acc_v2[0:1, :]
        c1 = acc_1[0:1, :]
        c2 = acc_2[0:1, :]
        c3 = acc_3[0:1, :]
        c4 = acc_4[0:1, :]
        flat1 = (acc_r[0:1, :] * _C + lane).astype(jnp.float32)
        flat2 = (acc_r2[0:1, :] * _C + lane).astype(jnp.float32)
        m1 = jnp.max(colv, axis=1, keepdims=True)
        sel = colv == m1
        pk = jnp.concatenate(
            [
                jnp.where(sel, -flat1, _NEG),
                jnp.where(sel, c1, _NEG),
                jnp.where(sel, c2, _NEG),
                jnp.where(sel, c3, _NEG),
                jnp.where(sel, c4, _NEG),
                jnp.where(sel, flat1, _NEG),
                jnp.where(sel, flat1, _NEG),
                jnp.where(sel, flat1, _NEG),
            ],
            axis=0,
        )
        r8 = jnp.max(pk, axis=1, keepdims=True)
        negf = r8[0:1, :]
        fx1 = r8[1:2, :]
        fy1 = r8[2:3, :]
        fx2 = r8[3:4, :]
        fy2 = r8[4:5, :]
        posf = r8[5:6, :]
        mixv = jnp.where(sel, colv2, colv)
        m2 = jnp.max(mixv, axis=1, keepdims=True)
        sel2 = mixv == m2
        mixf = jnp.where(sel, flat2, flat1)
        pk2 = jnp.concatenate(
            [
                jnp.where(sel2, -mixf, _NEG),
                jnp.where(sel2 & (~sel), c1, _NEG),
                jnp.where(sel2 & (~sel), c2, _NEG),
                jnp.where(sel2 & (~sel), c3, _NEG),
                jnp.where(sel2 & (~sel), c4, _NEG),
                jnp.where(sel2, mixf, _NEG),
                jnp.where(sel2, mixf, _NEG),
                jnp.where(sel2, mixf, _NEG),
            ],
            axis=0,
        )
        q8 = jnp.max(pk2, axis=1, keepdims=True)
        negf2 = q8[0:1, :]
        sx1 = q8[1:2, :]
        sy1 = q8[2:3, :]
        sx2 = q8[3:4, :]
        sy2 = q8[4:5, :]
        posf2 = q8[5:6, :]
        sel2u = sel2 & (~sel)
        mix2v = jnp.where(sel | sel2u, colv2, colv)
        m3 = jnp.max(mix2v, axis=1, keepdims=True)
        sel3 = mix2v == m3
        mix2f = jnp.where(sel | sel2u, flat2, flat1)
        fresh3 = sel3 & (~sel) & (~sel2u)
        pk3 = jnp.concatenate(
            [
                jnp.where(sel3, -mix2f, _NEG),
                jnp.where(fresh3, c1, _NEG),
                jnp.where(fresh3, c2, _NEG),
                jnp.where(fresh3, c3, _NEG),
                jnp.where(fresh3, c4, _NEG),
                jnp.where(sel3, mix2f, _NEG),
                jnp.where(sel3, mix2f, _NEG),
                jnp.where(sel3, mix2f, _NEG),
            ],
            axis=0,
        )
        t8 = jnp.max(pk3, axis=1, keepdims=True)
        negf3 = t8[0:1, :]
        tx1 = t8[1:2, :]
        ty1 = t8[2:3, :]
        tx2 = t8[3:4, :]
        ty2 = t8[4:5, :]
        posf3 = t8[5:6, :]
        tie1v = (posf + negf) != 0.0
        tie2v = (posf2 + negf2) != 0.0
        tie3v = (posf3 + negf3) != 0.0
        wlane = -negf - jnp.floor(-negf / _C) * _C
        slane_ = -negf2 - jnp.floor(-negf2 / _C) * _C
        tlane_ = -negf3 - jnp.floor(-negf3 / _C) * _C
        samelane = wlane == slane_
        iou12 = iou_of(sx1, sy1, sx2, sy2, fx1, fy1, fx2, fy2)
        sup2 = iou12 > _NMS_THRESH
        bad2v = tie1v | tie2v | samelane | (m2 == _NEG) | sup2
        stale3 = (tlane_ == wlane) | (tlane_ == slane_)
        sup31 = iou_of(tx1, ty1, tx2, ty2, fx1, fy1, fx2, fy2) > _NMS_THRESH
        sup32 = iou_of(tx1, ty1, tx2, ty2, sx1, sy1, sx2, sy2) > _NMS_THRESH
        bad3v = bad2v | tie3v | stale3 | (m3 == _NEG) | sup31 | sup32
        code = (
            jnp.where(tie1v, 1.0, 0.0)
            + jnp.where(bad2v, 2.0, 0.0)
            + jnp.where(bad3v, 4.0, 0.0)
        )
        code_s = code[0, 0]
        tie1_s = jnp.mod(code_s, 2.0) == 1.0
        accept2_s = (jnp.mod(code_s, 4.0) < 2.0) & (n_out < _MAX_DET - 1)
        accept3_s = (code_s < 4.0) & (n_out < _MAX_DET - 2)

        def fixco():
            um = flat1 == -negf
            fpk = jnp.concatenate(
                [
                    jnp.where(um, c1, _NEG),
                    jnp.where(um, c2, _NEG),
                    jnp.where(um, c3, _NEG),
                    jnp.where(um, c4, _NEG),
                ]
                + [jnp.where(um, c4, _NEG)] * 4,
                axis=0,
            )
            f8 = jnp.max(fpk, axis=1, keepdims=True)
            return f8[0:1, :], f8[1:2, :], f8[2:3, :], f8[3:4, :]

        wx1, wy1, wx2, wy2 = jax.lax.cond(
            tie1_s, fixco, lambda: (fx1, fy1, fx2, fy2)
        )
        valid_n = m1 != _NEG
        rowv1 = (
            jnp.where(lane == 0, wx1, 0.0)
            + jnp.where(lane == 1, wy1, 0.0)
            + jnp.where(lane == 2, wx2, 0.0)
            + jnp.where(lane == 3, wy2, 0.0)
            + jnp.where(lane == 4, m1, 0.0)
        )
        out_ref[pl.ds(n_out, 1), :] = jnp.where(valid_n, rowv1, 0.0)

        @pl.when(accept2_s)
        def _():
            rowv2 = (
                jnp.where(lane == 0, sx1, 0.0)
                + jnp.where(lane == 1, sy1, 0.0)
                + jnp.where(lane == 2, sx2, 0.0)
                + jnp.where(lane == 3, sy2, 0.0)
                + jnp.where(lane == 4, m2, 0.0)
            )
            out_ref[pl.ds(n_out + 1, 1), :] = rowv2

        @pl.when(accept3_s)
        def _():
            rowv3 = (
                jnp.where(lane == 0, tx1, 0.0)
                + jnp.where(lane == 1, ty1, 0.0)
                + jnp.where(lane == 2, tx2, 0.0)
                + jnp.where(lane == 3, ty2, 0.0)
                + jnp.where(lane == 4, m3, 0.0)
            )
            out_ref[pl.ds(n_out + 2, 1), :] = rowv3

        n_next = (
            n_out
            + 1
            + jnp.where(accept2_s, 1, 0).astype(jnp.int32)
            + jnp.where(accept3_s, 1, 0).astype(jnp.int32)
        )
        nb2v = jnp.where(accept2_s, m2, _NEG)
        nb3v = jnp.where(accept3_s, m3, _NEG)
        return (n_next, m1, wx1, wy1, wx2, wy2, nb2v, sx1, sy1, sx2, sy2,
                nb3v, tx1, ty1, tx2, ty2)

    z = jnp.zeros((1, 1), dtype=jnp.float32)
    ninf = jnp.full((1, 1), _NEG, dtype=jnp.float32)
    jax.lax.while_loop(
        lambda c: c[0] < _MAX_DET,
        round_body,
        (jnp.int32(0), ninf, z, z, z, z, ninf, z, z, z, z, ninf, z, z, z, z),
    )


def kernel(boxes, scores):
    pad_boxes = jnp.zeros((_PAD - _N, 4), dtype=boxes.dtype)
    b = jnp.concatenate([boxes, pad_boxes], axis=0)
    s = jnp.concatenate(
        [scores, jnp.full((_PAD - _N,), -1.0, dtype=scores.dtype)], axis=0
    ).reshape(_R, _C)
    x1 = b[:, 0].reshape(_R, _C)
    y1 = b[:, 1].reshape(_R, _C)
    x2 = b[:, 2].reshape(_R, _C)
    y2 = b[:, 3].reshape(_R, _C)
    out = pl.pallas_call(
        _nms_kernel,
        out_shape=jax.ShapeDtypeStruct((_MAX_DET, _C), jnp.float32),
        scratch_shapes=[
            pltpu.VMEM((_R, _C), jnp.float32),
            pltpu.VMEM((_R, _C), jnp.float32),
        ],
    )(x1, y1, x2, y2, s)
    return out[:, :5]


# speedup vs baseline: 1.6277x; 1.0735x over previous
import jax
import jax.numpy as jnp
from jax.experimental import pallas as pl
from jax.experimental.pallas import tpu as pltpu

_SCORE_THRESH = 0.05
_NMS_THRESH = 0.5
_MAX_DET = 100
_N = 20000
_R = 160
_C = 128
_PAD = _R * _C
_NBLK = _R // 8
_NEG = -jnp.inf


def _nms_kernel(x1_ref, y1_ref, x2_ref, y2_ref, s_ref, out_ref, work_ref, area_ref):
    area_ref[...] = (x2_ref[...] - x1_ref[...]) * (y2_ref[...] - y1_ref[...])
    s = s_ref[...]
    work_ref[...] = jnp.where(s > _SCORE_THRESH, s, _NEG)

    lane = jax.lax.broadcasted_iota(jnp.int32, (1, _C), 1)
    row8 = jax.lax.broadcasted_iota(jnp.int32, (8, _C), 0)
    lanef = lane.astype(jnp.float32)

    def iou_of(ax1, ay1, ax2, ay2, bx1, by1, bx2, by2):
        xx1 = jnp.maximum(ax1, bx1)
        yy1 = jnp.maximum(ay1, by1)
        xx2 = jnp.minimum(ax2, bx2)
        yy2 = jnp.minimum(ay2, by2)
        inter = jnp.maximum(xx2 - xx1, 0.0) * jnp.maximum(yy2 - yy1, 0.0)
        aarea = (ax2 - ax1) * (ay2 - ay1)
        barea = (bx2 - bx1) * (by2 - by1)
        return inter / (aarea + barea - inter + 1e-9)

    def round_body(carry):
        (n_out, b1v, p11, p12, p13, p14, b2v, p21, p22, p23, p24,
         b3v, p31, p32, p33, p34, b4v, p41, p42, p43, p44) = carry
        valid1 = b1v != _NEG
        valid2 = b2v != _NEG
        valid3 = b3v != _NEG
        valid4 = b4v != _NEG
        barea1 = (p13 - p11) * (p14 - p12)
        barea2 = (p23 - p21) * (p24 - p22)
        barea3 = (p33 - p31) * (p34 - p32)
        barea4 = (p43 - p41) * (p44 - p42)
        acc_v = jnp.full((8, _C), _NEG, dtype=jnp.float32)
        acc_r = jnp.zeros((8, _C), dtype=jnp.int32)
        acc_v2 = acc_v
        acc_r2 = acc_r
        acc_1 = jnp.zeros((8, _C), dtype=jnp.float32)
        acc_2 = acc_1
        acc_3 = acc_1
        acc_4 = acc_1
        for v in range(_NBLK):
            sl = pl.ds(v * 8, 8)
            xv1 = x1_ref[sl, :]
            yv1 = y1_ref[sl, :]
            xv2 = x2_ref[sl, :]
            yv2 = y2_ref[sl, :]
            wv = work_ref[sl, :]
            av = area_ref[sl, :]
            xxa = jnp.maximum(xv1, p11)
            yya = jnp.maximum(yv1, p12)
            xxb = jnp.minimum(xv2, p13)
            yyb = jnp.minimum(yv2, p14)
            ia = jnp.maximum(xxb - xxa, 0.0) * jnp.maximum(yyb - yya, 0.0)
            iou1 = ia / (av + barea1 - ia + 1e-9)
            sup = (iou1 > _NMS_THRESH) & valid1
            xxa = jnp.maximum(xv1, p21)
            yya = jnp.maximum(yv1, p22)
            xxb = jnp.minimum(xv2, p23)
            yyb = jnp.minimum(yv2, p24)
            ib = jnp.maximum(xxb - xxa, 0.0) * jnp.maximum(yyb - yya, 0.0)
            iou2 = ib / (av + barea2 - ib + 1e-9)
            sup = sup | ((iou2 > _NMS_THRESH) & valid2)
            xxa = jnp.maximum(xv1, p31)
            yya = jnp.maximum(yv1, p32)
            xxb = jnp.minimum(xv2, p33)
            yyb = jnp.minimum(yv2, p34)
            ic = jnp.maximum(xxb - xxa, 0.0) * jnp.maximum(yyb - yya, 0.0)
            iou3 = ic / (av + barea3 - ic + 1e-9)
            sup = sup | ((iou3 > _NMS_THRESH) & valid3)
            xxa = jnp.maximum(xv1, p41)
            yya = jnp.maximum(yv1, p42)
            xxb = jnp.minimum(xv2, p43)
            yyb = jnp.minimum(yv2, p44)
            idd = jnp.maximum(xxb - xxa, 0.0) * jnp.maximum(yyb - yya, 0.0)
            iou4 = idd / (av + barea4 - idd + 1e-9)
            sup = sup | ((iou4 > _NMS_THRESH) & valid4)
            nw = jnp.where(sup, _NEG, wv)
            work_ref[sl, :] = nw
            gt1 = nw > acc_v
            gt2 = (~gt1) & (nw > acc_v2)
            acc_v2 = jnp.where(gt1, acc_v, jnp.where(gt2, nw, acc_v2))
            acc_r2 = jnp.where(gt1, acc_r, jnp.where(gt2, row8 + v * 8, acc_r2))
            acc_r = jnp.where(gt1, row8 + v * 8, acc_r)
            acc_1 = jnp.where(gt1, xv1, acc_1)
            acc_2 = jnp.where(gt1, yv1, acc_2)
            acc_3 = jnp.where(gt1, xv2, acc_3)
            acc_4 = jnp.where(gt1, yv2, acc_4)
            acc_v = jnp.where(gt1, nw, acc_v)
        for k in (4, 2, 1):
            bv = pltpu.roll(acc_v, k, 0)
            br = pltpu.roll(acc_r, k, 0)
            bv2 = pltpu.roll(acc_v2, k, 0)
            br2 = pltpu.roll(acc_r2, k, 0)
            bfirst = (bv > acc_v) | ((bv == acc_v) & (br < acc_r))
            tx = (bv2 > acc_v) | ((bv2 == acc_v) & (br2 < acc_r))
            vx = jnp.where(tx, bv2, acc_v)
            rx = jnp.where(tx, br2, acc_r)
            ty = (bv > acc_v2) | ((bv == acc_v2) & (br < acc_r2))
            vy = jnp.where(ty, bv, acc_v2)
            ry = jnp.where(ty, br, acc_r2)
            acc_v2 = jnp.where(bfirst, vx, vy)
            acc_r2 = jnp.where(bfirst, rx, ry)
            acc_v = jnp.where(bfirst, bv, acc_v)
            acc_r = jnp.where(bfirst, br, acc_r)
            acc_1 = jnp.where(bfirst, pltpu.roll(acc_1, k, 0), acc_1)
            acc_2 = jnp.where(bfirst, pltpu.roll(acc_2, k, 0), acc_2)
            acc_3 = jnp.where(bfirst, pltpu.roll(acc_3, k, 0), acc_3)
            acc_4 = jnp.where(bfirst, pltpu.roll(acc_4, k, 0), acc_4)
        colv = acc_v[0:1, :]
        colv2 = acc_v2[0:1, :]
        c1 = acc_1[0:1, :]
        c2 = acc_2[0:1, :]
        c3 = acc_3[0:1, :]
        c4 = acc_4[0:1, :]
        flat1 = (acc_r[0:1, :] * _C + lane).astype(jnp.float32)
        flat2 = (acc_r2[0:1, :] * _C + lane).astype(jnp.float32)
        m1 = jnp.max(colv, axis=1, keepdims=True)
        sel = colv == m1
        pk = jnp.concatenate(
            [
                jnp.where(sel, -flat1, _NEG),
                jnp.where(sel, c1, _NEG),
                jnp.where(sel, c2, _NEG),
                jnp.where(sel, c3, _NEG),
                jnp.where(sel, c4, _NEG),
                jnp.where(sel, flat1, _NEG),
                jnp.where(sel, flat1, _NEG),
                jnp.where(sel, flat1, _NEG),
            ],
            axis=0,
        )
        r8 = jnp.max(pk, axis=1, keepdims=True)
        negf = r8[0:1, :]
        fx1 = r8[1:2, :]
        fy1 = r8[2:3, :]
        fx2 = r8[3:4, :]
        fy2 = r8[4:5, :]
        posf = r8[5:6, :]
        mixv = jnp.where(sel, colv2, colv)
        m2 = jnp.max(mixv, axis=1, keepdims=True)
        sel2 = mixv == m2
        mixf = jnp.where(sel, flat2, flat1)
        pk2 = jnp.concatenate(
            [
                jnp.where(sel2, -mixf, _NEG),
                jnp.where(sel2 & (~sel), c1, _NEG),
                jnp.where(sel2 & (~sel), c2, _NEG),
                jnp.where(sel2 & (~sel), c3, _NEG),
                jnp.where(sel2 & (~sel), c4, _NEG),
                jnp.where(sel2, mixf, _NEG),
                jnp.where(sel2, mixf, _NEG),
                jnp.where(sel2, mixf, _NEG),
            ],
            axis=0,
        )
        q8 = jnp.max(pk2, axis=1, keepdims=True)
        negf2 = q8[0:1, :]
        sx1 = q8[1:2, :]
        sy1 = q8[2:3, :]
        sx2 = q8[3:4, :]
        sy2 = q8[4:5, :]
        posf2 = q8[5:6, :]
        sel2u = sel2 & (~sel)
        mix2v = jnp.where(sel | sel2u, colv2, colv)
        m3 = jnp.max(mix2v, axis=1, keepdims=True)
        sel3 = mix2v == m3
        mix2f = jnp.where(sel | sel2u, flat2, flat1)
        fresh3 = sel3 & (~sel) & (~sel2u)
        pk3 = jnp.concatenate(
            [
                jnp.where(sel3, -mix2f, _NEG),
                jnp.where(fresh3, c1, _NEG),
                jnp.where(fresh3, c2, _NEG),
                jnp.where(fresh3, c3, _NEG),
                jnp.where(fresh3, c4, _NEG),
                jnp.where(sel3, mix2f, _NEG),
                jnp.where(sel3, mix2f, _NEG),
                jnp.where(sel3, mix2f, _NEG),
            ],
            axis=0,
        )
        t8 = jnp.max(pk3, axis=1, keepdims=True)
        negf3 = t8[0:1, :]
        tx1 = t8[1:2, :]
        ty1 = t8[2:3, :]
        tx2 = t8[3:4, :]
        ty2 = t8[4:5, :]
        posf3 = t8[5:6, :]
        sel3u = sel3 & (~sel) & (~sel2u)
        mix3v = jnp.where(sel | sel2u | sel3u, colv2, colv)
        m4 = jnp.max(mix3v, axis=1, keepdims=True)
        sel4 = mix3v == m4
        mix3f = jnp.where(sel | sel2u | sel3u, flat2, flat1)
        fresh4 = sel4 & (~sel) & (~sel2u) & (~sel3u)
        pk4 = jnp.concatenate(
            [
                jnp.where(sel4, -mix3f, _NEG),
                jnp.where(fresh4, c1, _NEG),
                jnp.where(fresh4, c2, _NEG),
                jnp.where(fresh4, c3, _NEG),
                jnp.where(fresh4, c4, _NEG),
                jnp.where(sel4, mix3f, _NEG),
                jnp.where(sel4, mix3f, _NEG),
                jnp.where(sel4, mix3f, _NEG),
            ],
            axis=0,
        )
        u8 = jnp.max(pk4, axis=1, keepdims=True)
        negf4 = u8[0:1, :]
        ux1 = u8[1:2, :]
        uy1 = u8[2:3, :]
        ux2 = u8[3:4, :]
        uy2 = u8[4:5, :]
        posf4 = u8[5:6, :]
        tie1v = (posf + negf) != 0.0
        tie2v = (posf2 + negf2) != 0.0
        tie3v = (posf3 + negf3) != 0.0
        wlane = -negf - jnp.floor(-negf / _C) * _C
        slane_ = -negf2 - jnp.floor(-negf2 / _C) * _C
        tlane_ = -negf3 - jnp.floor(-negf3 / _C) * _C
        samelane = wlane == slane_
        iou12 = iou_of(sx1, sy1, sx2, sy2, fx1, fy1, fx2, fy2)
        sup2 = iou12 > _NMS_THRESH
        bad2v = tie1v | tie2v | samelane | (m2 == _NEG) | sup2
        stale3 = (tlane_ == wlane) | (tlane_ == slane_)
        sup31 = iou_of(tx1, ty1, tx2, ty2, fx1, fy1, fx2, fy2) > _NMS_THRESH
        sup32 = iou_of(tx1, ty1, tx2, ty2, sx1, sy1, sx2, sy2) > _NMS_THRESH
        bad3v = bad2v | tie3v | stale3 | (m3 == _NEG) | sup31 | sup32
        tie4v = (posf4 + negf4) != 0.0
        ulane_ = -negf4 - jnp.floor(-negf4 / _C) * _C
        stale4 = (ulane_ == wlane) | (ulane_ == slane_) | (ulane_ == tlane_)
        sup41 = iou_of(ux1, uy1, ux2, uy2, fx1, fy1, fx2, fy2) > _NMS_THRESH
        sup42 = iou_of(ux1, uy1, ux2, uy2, sx1, sy1, sx2, sy2) > _NMS_THRESH
        sup43 = iou_of(ux1, uy1, ux2, uy2, tx1, ty1, tx2, ty2) > _NMS_THRESH
        bad4v = (
            bad3v | tie4v | stale4 | (m4 == _NEG) | sup41 | sup42 | sup43
        )
        code = (
            jnp.where(tie1v, 1.0, 0.0)
            + jnp.where(bad2v, 2.0, 0.0)
            + jnp.where(bad3v, 4.0, 0.0)
            + jnp.where(bad4v, 8.0, 0.0)
        )
        code_s = code[0, 0]
        tie1_s = jnp.mod(code_s, 2.0) == 1.0
        accept2_s = (jnp.mod(code_s, 4.0) < 2.0) & (n_out < _MAX_DET - 1)
        accept3_s = (jnp.mod(code_s, 8.0) < 4.0) & (n_out < _MAX_DET - 2)
        accept4_s = (code_s < 8.0) & (n_out < _MAX_DET - 3)

        def fixco():
            um = flat1 == -negf
            fpk = jnp.concatenate(
                [
                    jnp.where(um, c1, _NEG),
                    jnp.where(um, c2, _NEG),
                    jnp.where(um, c3, _NEG),
                    jnp.where(um, c4, _NEG),
                ]
                + [jnp.where(um, c4, _NEG)] * 4,
                axis=0,
            )
            f8 = jnp.max(fpk, axis=1, keepdims=True)
            return f8[0:1, :], f8[1:2, :], f8[2:3, :], f8[3:4, :]

        wx1, wy1, wx2, wy2 = jax.lax.cond(
            tie1_s, fixco, lambda: (fx1, fy1, fx2, fy2)
        )
        valid_n = m1 != _NEG
        rowv1 = (
            jnp.where(lane == 0, wx1, 0.0)
            + jnp.where(lane == 1, wy1, 0.0)
            + jnp.where(lane == 2, wx2, 0.0)
            + jnp.where(lane == 3, wy2, 0.0)
            + jnp.where(lane == 4, m1, 0.0)
        )
        out_ref[pl.ds(n_out, 1), :] = jnp.where(valid_n, rowv1, 0.0)

        @pl.when(accept2_s)
        def _():
            rowv2 = (
                jnp.where(lane == 0, sx1, 0.0)
                + jnp.where(lane == 1, sy1, 0.0)
                + jnp.where(lane == 2, sx2, 0.0)
                + jnp.where(lane == 3, sy2, 0.0)
                + jnp.where(lane == 4, m2, 0.0)
            )
            out_ref[pl.ds(n_out + 1, 1), :] = rowv2

        @pl.when(accept3_s)
        def _():
            rowv3 = (
                jnp.where(lane == 0, tx1, 0.0)
                + jnp.where(lane == 1, ty1, 0.0)
                + jnp.where(lane == 2, tx2, 0.0)
                + jnp.where(lane == 3, ty2, 0.0)
                + jnp.where(lane == 4, m3, 0.0)
            )
            out_ref[pl.ds(n_out + 2, 1), :] = rowv3

        @pl.when(accept4_s)
        def _():
            rowv4 = (
                jnp.where(lane == 0, ux1, 0.0)
                + jnp.where(lane == 1, uy1, 0.0)
                + jnp.where(lane == 2, ux2, 0.0)
                + jnp.where(lane == 3, uy2, 0.0)
                + jnp.where(lane == 4, m4, 0.0)
            )
            out_ref[pl.ds(n_out + 3, 1), :] = rowv4

        n_next = (
            n_out
            + 1
            + jnp.where(accept2_s, 1, 0).astype(jnp.int32)
            + jnp.where(accept3_s, 1, 0).astype(jnp.int32)
            + jnp.where(accept4_s, 1, 0).astype(jnp.int32)
        )
        nb2v = jnp.where(accept2_s, m2, _NEG)
        nb3v = jnp.where(accept3_s, m3, _NEG)
        nb4v = jnp.where(accept4_s, m4, _NEG)
        return (n_next, m1, wx1, wy1, wx2, wy2, nb2v, sx1, sy1, sx2, sy2,
                nb3v, tx1, ty1, tx2, ty2, nb4v, ux1, uy1, ux2, uy2)

    z = jnp.zeros((1, 1), dtype=jnp.float32)
    ninf = jnp.full((1, 1), _NEG, dtype=jnp.float32)
    jax.lax.while_loop(
        lambda c: c[0] < _MAX_DET,
        round_body,
        (jnp.int32(0), ninf, z, z, z, z, ninf, z, z, z, z,
         ninf, z, z, z, z, ninf, z, z, z, z),
    )


def kernel(boxes, scores):
    pad_boxes = jnp.zeros((_PAD - _N, 4), dtype=boxes.dtype)
    b = jnp.concatenate([boxes, pad_boxes], axis=0)
    s = jnp.concatenate(
        [scores, jnp.full((_PAD - _N,), -1.0, dtype=scores.dtype)], axis=0
    ).reshape(_R, _C)
    x1 = b[:, 0].reshape(_R, _C)
    y1 = b[:, 1].reshape(_R, _C)
    x2 = b[:, 2].reshape(_R, _C)
    y2 = b[:, 3].reshape(_R, _C)
    out = pl.pallas_call(
        _nms_kernel,
        out_shape=jax.ShapeDtypeStruct((_MAX_DET, _C), jnp.float32),
        scratch_shapes=[
            pltpu.VMEM((_R, _C), jnp.float32),
            pltpu.VMEM((_R, _C), jnp.float32),
        ],
    )(x1, y1, x2, y2, s)
    return out[:, :5]
